# baseline (speedup 1.0000x reference)
import functools

import jax
import jax.numpy as jnp
from jax import lax
from jax.experimental import pallas as pl
from jax.experimental.pallas import tpu as pltpu
from jax.experimental.pallas import tpu_sc as plsc

_N = 16384
_H = 64
_D = 128
_F = 3
_VT = 12500
_V2 = 50000

_NC = 2
_NS = 16
_NW = _NC * _NS
_BPW = _N // _NW
_CL = 128
_KCH = _BPW // _CL

_CW = 25
_TPW = _VT // _CW
_TCH = 10
_NIT = _TPW // (2 * _TCH)

_BT = 1024


def _sc_compact(act3, res3):
    mesh = plsc.VectorSubcoreMesh(core_axis_name="c", subcore_axis_name="s")

    @functools.partial(
        pl.kernel,
        mesh=mesh,
        out_type=[
            jax.ShapeDtypeStruct((_V2, _D), jnp.float32),
            jax.ShapeDtypeStruct((_V2, _D), jnp.float32),
        ],
        scratch_types=[
            pltpu.VMEM((2, _TCH, 8, _H), jnp.float32),
            pltpu.VMEM((2, 4 * _TCH, _D), jnp.float32),
            pltpu.SemaphoreType.DMA,
            pltpu.SemaphoreType.DMA,
            pltpu.SemaphoreType.DMA,
        ],
    )
    def compact_k(act_t, res_t, act_c, res_c, buf, cbuf, g0, g1, ws):
        wid = lax.axis_index("s") * _NC + lax.axis_index("c")
        nit = jnp.where(wid < _CW, _NIT, 0)
        tbase = wid * _TPW
        gsem = [g0, g1]

        def repack(b):
            for i in range(_TCH):
                for r in range(8):
                    q = i * 8 + r
                    for k in range(0, _H, 16):
                        cbuf[b, q // 2, (q % 2) * _H + k:(q % 2) * _H + k + 16] = \
                            buf[b, i, r, k:k + 16]

        def run(tab, out_c):
            def body(it, carry):
                t0 = tbase + it * 2 * _TCH
                g = [
                    pltpu.async_copy(tab.at[pl.ds(t0, _TCH)],
                                     buf.at[0], gsem[0]),
                    pltpu.async_copy(tab.at[pl.ds(t0 + _TCH, _TCH)],
                                     buf.at[1], gsem[1]),
                ]
                wd = []
                for b in range(2):
                    g[b].wait()
                    repack(b)
                    o0 = (t0 + b * _TCH) * 4
                    wd.append(pltpu.async_copy(
                        cbuf.at[b], out_c.at[pl.ds(o0, 4 * _TCH)], ws))
                for w in wd:
                    w.wait()
                return carry

            lax.fori_loop(0, nit, body, 0)

        run(act_t, act_c)
        run(res_t, res_c)

    return compact_k(act3, res3)


def _sc_gather(act2, res2, aidx, ridx):
    mesh = plsc.VectorSubcoreMesh(core_axis_name="c", subcore_axis_name="s")

    @functools.partial(
        pl.kernel,
        mesh=mesh,
        out_type=[
            jax.ShapeDtypeStruct((_N, _D), jnp.float32),
            jax.ShapeDtypeStruct((_N, _D), jnp.float32),
        ],
        scratch_types=[
            pltpu.VMEM((_BPW,), jnp.int32),
            pltpu.VMEM((_BPW,), jnp.int32),
            pltpu.VMEM((2, _CL, _D), jnp.float32),
            pltpu.VMEM((2, _CL, _D), jnp.float32),
            pltpu.SemaphoreType.DMA,
            pltpu.SemaphoreType.DMA,
            pltpu.SemaphoreType.DMA,
            pltpu.SemaphoreType.DMA,
        ],
    )
    def gather_k(act_t, res_t, aidx_h, ridx_h, act_o, res_o,
                 aidx_v, ridx_v, abuf, rbuf, g0, g1, w0, w1):
        wid = lax.axis_index("s") * _NC + lax.axis_index("c")
        base = wid * _BPW
        pltpu.sync_copy(aidx_h.at[pl.ds(base, _BPW)], aidx_v)
        pltpu.sync_copy(ridx_h.at[pl.ds(base, _BPW)], ridx_v)
        gsem = [g0, g1]
        wsem = [w0, w1]

        def fire_gather(j):
            b = j % 2
            ix = pl.ds(j * _CL, _CL)
            return [
                pltpu.async_copy(act_t.at[aidx_v.at[ix]], abuf.at[b], gsem[b]),
                pltpu.async_copy(res_t.at[ridx_v.at[ix]], rbuf.at[b], gsem[b]),
            ]

        def fire_write(j):
            b = j % 2
            ox = pl.ds(base + j * _CL, _CL)
            return [
                pltpu.async_copy(abuf.at[b], act_o.at[ox], wsem[b]),
                pltpu.async_copy(rbuf.at[b], res_o.at[ox], wsem[b]),
            ]

        gd = {0: fire_gather(0)}
        wd = {}
        for j in range(_KCH):
            if j + 1 < _KCH:
                if j - 1 >= 0:
                    for c in wd[j - 1]:
                        c.wait()
                gd[j + 1] = fire_gather(j + 1)
            for c in gd[j]:
                c.wait()
            wd[j] = fire_write(j)
        for j in (_KCH - 2, _KCH - 1):
            for c in wd[j]:
                c.wait()

    return gather_k(act2, res2, aidx, ridx)


def _ln_rows(x, g, b, eps=1e-5):
    mu = jnp.mean(x, axis=-1, keepdims=True)
    var = jnp.mean((x - mu) ** 2, axis=-1, keepdims=True)
    return (x - mu) / jnp.sqrt(var + eps) * g + b


def _film_body(nm_ref, nlg_ref, nlb_ref, w1_ref, b1_ref, mlg_ref, mlb_ref,
               wg_ref, bg_ref, wb_ref, bb_ref,
               gam_ref, bet_ref, ne_ref):
    f32 = jnp.float32
    nf = jnp.log1p(jnp.maximum(nm_ref[...], 0.0))
    nf = _ln_rows(nf, nlg_ref[...], nlb_ref[...])
    h = jnp.maximum(
        jnp.dot(nf, w1_ref[...], preferred_element_type=f32) + b1_ref[...], 0.0)
    num_emb = _ln_rows(h, mlg_ref[...], mlb_ref[...])
    gam_ref[...] = jax.nn.sigmoid(
        jnp.dot(num_emb, wg_ref[...], preferred_element_type=f32) + bg_ref[...])
    bet_ref[...] = (
        jnp.dot(num_emb, wb_ref[...], preferred_element_type=f32) + bb_ref[...])
    ne_ref[...] = num_emb


def _tc_film(num_arr, num_ln_g, num_ln_b, W1, b1, mlp_ln_g, mlp_ln_b,
             Wg, bg, Wb, bb):
    grid = (_N // _BT,)
    row = lambda i: (i, 0)
    full1 = lambda i: (0,)
    full2 = lambda i: (0, 0)
    in_specs = [
        pl.BlockSpec((_BT, _F), row),
        pl.BlockSpec((_F,), full1),
        pl.BlockSpec((_F,), full1),
        pl.BlockSpec((_F, _D), full2),
        pl.BlockSpec((_D,), full1),
        pl.BlockSpec((_D,), full1),
        pl.BlockSpec((_D,), full1),
        pl.BlockSpec((_D, _D), full2),
        pl.BlockSpec((_D,), full1),
        pl.BlockSpec((_D, _D), full2),
        pl.BlockSpec((_D,), full1),
    ]
    shp = jax.ShapeDtypeStruct((_N, _D), jnp.float32)
    return pl.pallas_call(
        _film_body,
        grid=grid,
        in_specs=in_specs,
        out_specs=[pl.BlockSpec((_BT, _D), row)] * 3,
        out_shape=[shp, shp, shp],
        compiler_params=pltpu.CompilerParams(
            dimension_semantics=("parallel",)),
    )(num_arr, num_ln_g, num_ln_b, W1, b1, mlp_ln_g, mlp_ln_b, Wg, bg, Wb, bb)


def _mod_body(a_ref, r_ref, ga_ref, gr_ref, gam_ref, bet_ref, ne_ref,
              wp_ref, bp_ref, plg_ref, plb_ref, out_ref):
    f32 = jnp.float32
    a = a_ref[...]
    r = r_ref[...]
    pa = (a & 1) == 1
    pr = (r & 1) == 1
    ga = ga_ref[...]
    gr = gr_ref[...]
    ah = jnp.where(pa, ga[:, _H:], ga[:, :_H])
    rh = jnp.where(pr, gr[:, _H:], gr[:, :_H])
    cat = jnp.concatenate([ah, rh], axis=-1)
    cat_mod = cat * gam_ref[...] + bet_ref[...]
    is_pad = (a == 0) & (r == 0)
    cat_mod = jnp.where(is_pad, 0.0, cat_mod)
    num_emb = jnp.where(is_pad, 0.0, ne_ref[...])
    pre = (jnp.dot(cat_mod, wp_ref[0:_D, :], preferred_element_type=f32)
           + jnp.dot(num_emb, wp_ref[_D:2 * _D, :], preferred_element_type=f32)
           + bp_ref[...])
    out_ref[...] = _ln_rows(jnp.maximum(pre, 0.0), plg_ref[...], plb_ref[...])


def _tc_mod(acts2, ress2, act_rows, res_rows, gamma, beta, num_emb,
            Wp, bp, proj_ln_g, proj_ln_b):
    grid = (_N // _BT,)
    row = lambda i: (i, 0)
    full1 = lambda i: (0,)
    full2 = lambda i: (0, 0)
    in_specs = [
        pl.BlockSpec((_BT, 1), row),
        pl.BlockSpec((_BT, 1), row),
        pl.BlockSpec((_BT, _D), row),
        pl.BlockSpec((_BT, _D), row),
        pl.BlockSpec((_BT, _D), row),
        pl.BlockSpec((_BT, _D), row),
        pl.BlockSpec((_BT, _D), row),
        pl.BlockSpec((2 * _D, _D), full2),
        pl.BlockSpec((_D,), full1),
        pl.BlockSpec((_D,), full1),
        pl.BlockSpec((_D,), full1),
    ]
    return pl.pallas_call(
        _mod_body,
        grid=grid,
        in_specs=in_specs,
        out_specs=pl.BlockSpec((_BT, _D), row),
        out_shape=jax.ShapeDtypeStruct((_N, _D), jnp.float32),
        compiler_params=pltpu.CompilerParams(
            dimension_semantics=("parallel",)),
    )(acts2, ress2, act_rows, res_rows, gamma, beta, num_emb,
      Wp, bp, proj_ln_g, proj_ln_b)


def kernel(activities, resources, num_arr, act_table, res_table,
           num_ln_g, num_ln_b, W1, b1, mlp_ln_g, mlp_ln_b,
           Wg, bg, Wb, bb, Wp, bp, proj_ln_g, proj_ln_b):
    acts = activities.astype(jnp.int32)
    ress = resources.astype(jnp.int32)
    gamma, beta, num_emb = _tc_film(
        num_arr, num_ln_g, num_ln_b, W1, b1, mlp_ln_g, mlp_ln_b, Wg, bg, Wb, bb)
    act2, res2 = _sc_compact(act_table.reshape(_VT, 8, _H),
                             res_table.reshape(_VT, 8, _H))
    act_rows, res_rows = _sc_gather(act2, res2, acts >> 1, ress >> 1)
    return _tc_mod(acts.reshape(_N, 1), ress.reshape(_N, 1),
                   act_rows, res_rows, gamma, beta, num_emb,
                   Wp, bp, proj_ln_g, proj_ln_b)


# speedup vs baseline: 1.2769x; 1.2769x over previous
import functools

import jax
import jax.numpy as jnp
from jax import lax
from jax.experimental import pallas as pl
from jax.experimental.pallas import tpu as pltpu
from jax.experimental.pallas import tpu_sc as plsc

_N = 16384
_H = 64
_D = 128
_F = 3

_NC = 2
_NS = 16
_NW = _NC * _NS
_BPW = _N // _NW
_CL = 128
_KCH = _BPW // _CL

_BT = 1024


def _sc_gather(act_table, res_table, aidx, ridx):
    mesh = plsc.VectorSubcoreMesh(core_axis_name="c", subcore_axis_name="s")

    @functools.partial(
        pl.kernel,
        mesh=mesh,
        out_type=[
            jax.ShapeDtypeStruct((_N, _H), jnp.float32),
            jax.ShapeDtypeStruct((_N, _H), jnp.float32),
        ],
        scratch_types=[
            pltpu.VMEM((_BPW,), jnp.int32),
            pltpu.VMEM((_BPW,), jnp.int32),
            pltpu.VMEM((2, _CL, _H), jnp.float32),
            pltpu.VMEM((2, _CL, _H), jnp.float32),
            pltpu.SemaphoreType.DMA,
            pltpu.SemaphoreType.DMA,
            pltpu.SemaphoreType.DMA,
            pltpu.SemaphoreType.DMA,
        ],
        compiler_params=pltpu.CompilerParams(use_tc_tiling_on_sc=False),
    )
    def gather_k(act_t, res_t, aidx_h, ridx_h, act_o, res_o,
                 aidx_v, ridx_v, abuf, rbuf, g0, g1, w0, w1):
        wid = lax.axis_index("s") * _NC + lax.axis_index("c")
        base = wid * _BPW
        pltpu.sync_copy(aidx_h.at[pl.ds(base, _BPW)], aidx_v)
        pltpu.sync_copy(ridx_h.at[pl.ds(base, _BPW)], ridx_v)
        gsem = [g0, g1]
        wsem = [w0, w1]

        def fire_gather(j):
            b = j % 2
            ix = pl.ds(j * _CL, _CL)
            return [
                pltpu.async_copy(act_t.at[aidx_v.at[ix]], abuf.at[b], gsem[b]),
                pltpu.async_copy(res_t.at[ridx_v.at[ix]], rbuf.at[b], gsem[b]),
            ]

        def fire_write(j):
            b = j % 2
            ox = pl.ds(base + j * _CL, _CL)
            return [
                pltpu.async_copy(abuf.at[b], act_o.at[ox], wsem[b]),
                pltpu.async_copy(rbuf.at[b], res_o.at[ox], wsem[b]),
            ]

        gd = {0: fire_gather(0)}
        wd = {}
        for j in range(_KCH):
            if j + 1 < _KCH:
                if j - 1 >= 0:
                    for c in wd[j - 1]:
                        c.wait()
                gd[j + 1] = fire_gather(j + 1)
            for c in gd[j]:
                c.wait()
            wd[j] = fire_write(j)
        for j in (_KCH - 2, _KCH - 1):
            for c in wd[j]:
                c.wait()

    return gather_k(act_table, res_table, aidx, ridx)


def _ln_rows(x, g, b, eps=1e-5):
    mu = jnp.mean(x, axis=-1, keepdims=True)
    var = jnp.mean((x - mu) ** 2, axis=-1, keepdims=True)
    return (x - mu) / jnp.sqrt(var + eps) * g + b


def _dense_body(a_ref, r_ref, nm_ref, ae_ref, re_ref,
                nlg_ref, nlb_ref, w1_ref, b1_ref, mlg_ref, mlb_ref,
                wg_ref, bg_ref, wb_ref, bb_ref, wp_ref, bp_ref,
                plg_ref, plb_ref, out_ref):
    f32 = jnp.float32
    cat = jnp.concatenate([ae_ref[...], re_ref[...]], axis=-1)
    nf = jnp.log1p(jnp.maximum(nm_ref[...], 0.0))
    nf = _ln_rows(nf, nlg_ref[...], nlb_ref[...])
    h = jnp.maximum(
        jnp.dot(nf, w1_ref[...], preferred_element_type=f32) + b1_ref[...], 0.0)
    num_emb = _ln_rows(h, mlg_ref[...], mlb_ref[...])
    gamma = jax.nn.sigmoid(
        jnp.dot(num_emb, wg_ref[...], preferred_element_type=f32) + bg_ref[...])
    beta = jnp.dot(num_emb, wb_ref[...], preferred_element_type=f32) + bb_ref[...]
    cat_mod = cat * gamma + beta
    is_pad = (a_ref[...] == 0) & (r_ref[...] == 0)
    cat_mod = jnp.where(is_pad, 0.0, cat_mod)
    num_emb = jnp.where(is_pad, 0.0, num_emb)
    pre = (jnp.dot(cat_mod, wp_ref[0:_D, :], preferred_element_type=f32)
           + jnp.dot(num_emb, wp_ref[_D:2 * _D, :], preferred_element_type=f32)
           + bp_ref[...])
    out_ref[...] = _ln_rows(jnp.maximum(pre, 0.0), plg_ref[...], plb_ref[...])


def _tc_dense(acts2, ress2, num_arr, act_emb, res_emb,
              num_ln_g, num_ln_b, W1, b1, mlp_ln_g, mlp_ln_b,
              Wg, bg, Wb, bb, Wp, bp, proj_ln_g, proj_ln_b):
    grid = (_N // _BT,)
    row = lambda i: (i, 0)
    full1 = lambda i: (0,)
    full2 = lambda i: (0, 0)
    in_specs = [
        pl.BlockSpec((_BT, 1), row),
        pl.BlockSpec((_BT, 1), row),
        pl.BlockSpec((_BT, _F), row),
        pl.BlockSpec((_BT, _H), row),
        pl.BlockSpec((_BT, _H), row),
        pl.BlockSpec((_F,), full1),
        pl.BlockSpec((_F,), full1),
        pl.BlockSpec((_F, _D), full2),
        pl.BlockSpec((_D,), full1),
        pl.BlockSpec((_D,), full1),
        pl.BlockSpec((_D,), full1),
        pl.BlockSpec((_D, _D), full2),
        pl.BlockSpec((_D,), full1),
        pl.BlockSpec((_D, _D), full2),
        pl.BlockSpec((_D,), full1),
        pl.BlockSpec((2 * _D, _D), full2),
        pl.BlockSpec((_D,), full1),
        pl.BlockSpec((_D,), full1),
        pl.BlockSpec((_D,), full1),
    ]
    return pl.pallas_call(
        _dense_body,
        grid=grid,
        in_specs=in_specs,
        out_specs=pl.BlockSpec((_BT, _D), row),
        out_shape=jax.ShapeDtypeStruct((_N, _D), jnp.float32),
        compiler_params=pltpu.CompilerParams(
            dimension_semantics=("parallel",)),
    )(acts2, ress2, num_arr, act_emb, res_emb,
      num_ln_g, num_ln_b, W1, b1, mlp_ln_g, mlp_ln_b,
      Wg, bg, Wb, bb, Wp, bp, proj_ln_g, proj_ln_b)


def kernel(activities, resources, num_arr, act_table, res_table,
           num_ln_g, num_ln_b, W1, b1, mlp_ln_g, mlp_ln_b,
           Wg, bg, Wb, bb, Wp, bp, proj_ln_g, proj_ln_b):
    acts = activities.astype(jnp.int32)
    ress = resources.astype(jnp.int32)
    act_emb, res_emb = _sc_gather(act_table, res_table, acts, ress)
    return _tc_dense(acts.reshape(_N, 1), ress.reshape(_N, 1), num_arr,
                     act_emb, res_emb,
                     num_ln_g, num_ln_b, W1, b1, mlp_ln_g, mlp_ln_b,
                     Wg, bg, Wb, bb, Wp, bp, proj_ln_g, proj_ln_b)


# speedup vs baseline: 1.2817x; 1.0038x over previous
import functools

import jax
import jax.numpy as jnp
from jax import lax
from jax.experimental import pallas as pl
from jax.experimental.pallas import tpu as pltpu
from jax.experimental.pallas import tpu_sc as plsc

_N = 16384
_H = 64
_D = 128
_F = 3

_NC = 2
_NS = 16
_NW = _NC * _NS
_BPW = _N // _NW
_CL = 128
_KCH = _BPW // _CL

_BT = 2048


def _sc_gather(act_table, res_table, aidx, ridx):
    mesh = plsc.VectorSubcoreMesh(core_axis_name="c", subcore_axis_name="s")

    @functools.partial(
        pl.kernel,
        mesh=mesh,
        out_type=[
            jax.ShapeDtypeStruct((_N, _H), jnp.float32),
            jax.ShapeDtypeStruct((_N, _H), jnp.float32),
        ],
        scratch_types=[
            pltpu.VMEM((_BPW,), jnp.int32),
            pltpu.VMEM((_BPW,), jnp.int32),
            pltpu.VMEM((2, _CL, _H), jnp.float32),
            pltpu.VMEM((2, _CL, _H), jnp.float32),
            pltpu.SemaphoreType.DMA,
            pltpu.SemaphoreType.DMA,
            pltpu.SemaphoreType.DMA,
            pltpu.SemaphoreType.DMA,
        ],
        compiler_params=pltpu.CompilerParams(use_tc_tiling_on_sc=False),
    )
    def gather_k(act_t, res_t, aidx_h, ridx_h, act_o, res_o,
                 aidx_v, ridx_v, abuf, rbuf, g0, g1, w0, w1):
        wid = lax.axis_index("s") * _NC + lax.axis_index("c")
        base = wid * _BPW
        pltpu.sync_copy(aidx_h.at[pl.ds(base, _BPW)], aidx_v)
        pltpu.sync_copy(ridx_h.at[pl.ds(base, _BPW)], ridx_v)
        gsem = [g0, g1]
        wsem = [w0, w1]

        def fire_gather(j):
            b = j % 2
            ix = pl.ds(j * _CL, _CL)
            return [
                pltpu.async_copy(act_t.at[aidx_v.at[ix]], abuf.at[b], gsem[b]),
                pltpu.async_copy(res_t.at[ridx_v.at[ix]], rbuf.at[b], gsem[b]),
            ]

        def fire_write(j):
            b = j % 2
            ox = pl.ds(base + j * _CL, _CL)
            return [
                pltpu.async_copy(abuf.at[b], act_o.at[ox], wsem[b]),
                pltpu.async_copy(rbuf.at[b], res_o.at[ox], wsem[b]),
            ]

        gd = {0: fire_gather(0)}
        wd = {}
        for j in range(_KCH):
            if j + 1 < _KCH:
                if j - 1 >= 0:
                    for c in wd[j - 1]:
                        c.wait()
                gd[j + 1] = fire_gather(j + 1)
            for c in gd[j]:
                c.wait()
            wd[j] = fire_write(j)
        for j in (_KCH - 2, _KCH - 1):
            for c in wd[j]:
                c.wait()

    return gather_k(act_table, res_table, aidx, ridx)


def _ln_rows(x, g, b, eps=1e-5):
    mu = jnp.mean(x, axis=-1, keepdims=True)
    var = jnp.mean((x - mu) ** 2, axis=-1, keepdims=True)
    return (x - mu) / jnp.sqrt(var + eps) * g + b


def _dense_body(a_ref, r_ref, nm_ref, ae_ref, re_ref,
                nlg_ref, nlb_ref, w1_ref, b1_ref, mlg_ref, mlb_ref,
                wg_ref, bg_ref, wb_ref, bb_ref, wp_ref, bp_ref,
                plg_ref, plb_ref, out_ref):
    f32 = jnp.float32
    cat = jnp.concatenate([ae_ref[...], re_ref[...]], axis=-1)
    nf = jnp.log1p(jnp.maximum(nm_ref[...], 0.0))
    nf = _ln_rows(nf, nlg_ref[...], nlb_ref[...])
    h = jnp.maximum(
        jnp.dot(nf, w1_ref[...], preferred_element_type=f32) + b1_ref[...], 0.0)
    num_emb = _ln_rows(h, mlg_ref[...], mlb_ref[...])
    gamma = jax.nn.sigmoid(
        jnp.dot(num_emb, wg_ref[...], preferred_element_type=f32) + bg_ref[...])
    beta = jnp.dot(num_emb, wb_ref[...], preferred_element_type=f32) + bb_ref[...]
    cat_mod = cat * gamma + beta
    is_pad = (a_ref[...] == 0) & (r_ref[...] == 0)
    cat_mod = jnp.where(is_pad, 0.0, cat_mod)
    num_emb = jnp.where(is_pad, 0.0, num_emb)
    pre = (jnp.dot(cat_mod, wp_ref[0:_D, :], preferred_element_type=f32)
           + jnp.dot(num_emb, wp_ref[_D:2 * _D, :], preferred_element_type=f32)
           + bp_ref[...])
    out_ref[...] = _ln_rows(jnp.maximum(pre, 0.0), plg_ref[...], plb_ref[...])


def _tc_dense(acts2, ress2, num_arr, act_emb, res_emb,
              num_ln_g, num_ln_b, W1, b1, mlp_ln_g, mlp_ln_b,
              Wg, bg, Wb, bb, Wp, bp, proj_ln_g, proj_ln_b):
    grid = (_N // _BT,)
    row = lambda i: (i, 0)
    full1 = lambda i: (0,)
    full2 = lambda i: (0, 0)
    in_specs = [
        pl.BlockSpec((_BT, 1), row),
        pl.BlockSpec((_BT, 1), row),
        pl.BlockSpec((_BT, _F), row),
        pl.BlockSpec((_BT, _H), row),
        pl.BlockSpec((_BT, _H), row),
        pl.BlockSpec((_F,), full1),
        pl.BlockSpec((_F,), full1),
        pl.BlockSpec((_F, _D), full2),
        pl.BlockSpec((_D,), full1),
        pl.BlockSpec((_D,), full1),
        pl.BlockSpec((_D,), full1),
        pl.BlockSpec((_D, _D), full2),
        pl.BlockSpec((_D,), full1),
        pl.BlockSpec((_D, _D), full2),
        pl.BlockSpec((_D,), full1),
        pl.BlockSpec((2 * _D, _D), full2),
        pl.BlockSpec((_D,), full1),
        pl.BlockSpec((_D,), full1),
        pl.BlockSpec((_D,), full1),
    ]
    return pl.pallas_call(
        _dense_body,
        grid=grid,
        in_specs=in_specs,
        out_specs=pl.BlockSpec((_BT, _D), row),
        out_shape=jax.ShapeDtypeStruct((_N, _D), jnp.float32),
        compiler_params=pltpu.CompilerParams(
            dimension_semantics=("parallel",)),
    )(acts2, ress2, num_arr, act_emb, res_emb,
      num_ln_g, num_ln_b, W1, b1, mlp_ln_g, mlp_ln_b,
      Wg, bg, Wb, bb, Wp, bp, proj_ln_g, proj_ln_b)


def kernel(activities, resources, num_arr, act_table, res_table,
           num_ln_g, num_ln_b, W1, b1, mlp_ln_g, mlp_ln_b,
           Wg, bg, Wb, bb, Wp, bp, proj_ln_g, proj_ln_b):
    acts = activities.astype(jnp.int32)
    ress = resources.astype(jnp.int32)
    act_emb, res_emb = _sc_gather(act_table, res_table, acts, ress)
    return _tc_dense(acts.reshape(_N, 1), ress.reshape(_N, 1), num_arr,
                     act_emb, res_emb,
                     num_ln_g, num_ln_b, W1, b1, mlp_ln_g, mlp_ln_b,
                     Wg, bg, Wb, bb, Wp, bp, proj_ln_g, proj_ln_b)


# speedup vs baseline: 1.2851x; 1.0027x over previous
import functools

import jax
import jax.numpy as jnp
from jax import lax
from jax.experimental import pallas as pl
from jax.experimental.pallas import tpu as pltpu
from jax.experimental.pallas import tpu_sc as plsc

_N = 16384
_H = 64
_D = 128
_F = 3

_NC = 2
_NS = 16
_NW = _NC * _NS
_BPW = _N // _NW
_CL = 128
_KCH = _BPW // _CL

_BT = 2048


def _sc_gather(act_table, res_table, aidx, ridx):
    mesh = plsc.VectorSubcoreMesh(core_axis_name="c", subcore_axis_name="s")

    @functools.partial(
        pl.kernel,
        mesh=mesh,
        out_type=[
            jax.ShapeDtypeStruct((_N, _H), jnp.float32),
            jax.ShapeDtypeStruct((_N, _H), jnp.float32),
        ],
        scratch_types=[
            pltpu.VMEM((_BPW,), jnp.int32),
            pltpu.VMEM((_BPW,), jnp.int32),
            pltpu.VMEM((2, _CL, _H), jnp.float32),
            pltpu.VMEM((2, _CL, _H), jnp.float32),
            pltpu.SemaphoreType.DMA,
            pltpu.SemaphoreType.DMA,
            pltpu.SemaphoreType.DMA,
            pltpu.SemaphoreType.DMA,
        ],
        compiler_params=pltpu.CompilerParams(use_tc_tiling_on_sc=False),
    )
    def gather_k(act_t, res_t, aidx_h, ridx_h, act_o, res_o,
                 aidx_v, ridx_v, abuf, rbuf, g0, g1, w0, w1):
        wid = lax.axis_index("s") * _NC + lax.axis_index("c")
        base = wid * _BPW
        pltpu.sync_copy(aidx_h.at[pl.ds(base, _BPW)], aidx_v)
        pltpu.sync_copy(ridx_h.at[pl.ds(base, _BPW)], ridx_v)
        gsem = [g0, g1]
        wsem = [w0, w1]

        def fire_gather(j):
            b = j % 2
            ix = pl.ds(j * _CL, _CL)
            return [
                pltpu.async_copy(act_t.at[aidx_v.at[ix]], abuf.at[b], gsem[b]),
                pltpu.async_copy(res_t.at[ridx_v.at[ix]], rbuf.at[b], gsem[b]),
            ]

        def fire_write(j):
            b = j % 2
            ox = pl.ds(base + j * _CL, _CL)
            return [
                pltpu.async_copy(abuf.at[b], act_o.at[ox], wsem[b]),
                pltpu.async_copy(rbuf.at[b], res_o.at[ox], wsem[b]),
            ]

        gd = {0: fire_gather(0)}
        wd = {}
        for j in range(_KCH):
            if j + 1 < _KCH:
                if j - 1 >= 0:
                    for c in wd[j - 1]:
                        c.wait()
                gd[j + 1] = fire_gather(j + 1)
            for c in gd[j]:
                c.wait()
            wd[j] = fire_write(j)
        for j in (_KCH - 2, _KCH - 1):
            for c in wd[j]:
                c.wait()

    return gather_k(act_table, res_table, aidx, ridx)


def _ln_rows(x, g, b, eps=1e-5):
    mu = jnp.mean(x, axis=-1, keepdims=True)
    var = jnp.mean((x - mu) ** 2, axis=-1, keepdims=True)
    return (x - mu) / jnp.sqrt(var + eps) * g + b


def _dense_body(a_ref, r_ref, nm_ref, ae_ref, re_ref,
                nlg_ref, nlb_ref, w1_ref, b1_ref, mlg_ref, mlb_ref,
                wg_ref, bg_ref, wb_ref, bb_ref, wp_ref, bp_ref,
                plg_ref, plb_ref, out_ref):
    f32 = jnp.float32
    is_pad = (a_ref[...] == 0) & (r_ref[...] == 0)
    cat = jnp.concatenate([ae_ref[...], re_ref[...]], axis=-1)
    nf = jnp.log1p(jnp.maximum(nm_ref[...], 0.0))
    nf = _ln_rows(nf, nlg_ref[...], nlb_ref[...])
    h = jnp.maximum(
        jnp.dot(nf, w1_ref[...], preferred_element_type=f32) + b1_ref[...], 0.0)
    num_emb = _ln_rows(h, mlg_ref[...], mlb_ref[...])
    gamma = jax.nn.sigmoid(
        jnp.dot(num_emb, wg_ref[...], preferred_element_type=f32) + bg_ref[...])
    beta = jnp.dot(num_emb, wb_ref[...], preferred_element_type=f32) + bb_ref[...]
    cat_mod = cat * gamma + beta
    cat_mod = jnp.where(is_pad, 0.0, cat_mod)
    num_emb = jnp.where(is_pad, 0.0, num_emb)
    pre = (jnp.dot(cat_mod, wp_ref[0:_D, :], preferred_element_type=f32)
           + jnp.dot(num_emb, wp_ref[_D:2 * _D, :], preferred_element_type=f32)
           + bp_ref[...])
    out_ref[...] = _ln_rows(jnp.maximum(pre, 0.0), plg_ref[...], plb_ref[...])


def _tc_dense(acts2, ress2, num_arr, act_emb, res_emb,
              num_ln_g, num_ln_b, W1, b1, mlp_ln_g, mlp_ln_b,
              Wg, bg, Wb, bb, Wp, bp, proj_ln_g, proj_ln_b):
    grid = (_N // _BT,)
    row = lambda i: (i, 0)
    full1 = lambda i: (0,)
    full2 = lambda i: (0, 0)
    in_specs = [
        pl.BlockSpec((_BT, 1), row),
        pl.BlockSpec((_BT, 1), row),
        pl.BlockSpec((_BT, _F), row),
        pl.BlockSpec((_BT, _H), row),
        pl.BlockSpec((_BT, _H), row),
        pl.BlockSpec((_F,), full1),
        pl.BlockSpec((_F,), full1),
        pl.BlockSpec((_F, _D), full2),
        pl.BlockSpec((_D,), full1),
        pl.BlockSpec((_D,), full1),
        pl.BlockSpec((_D,), full1),
        pl.BlockSpec((_D, _D), full2),
        pl.BlockSpec((_D,), full1),
        pl.BlockSpec((_D, _D), full2),
        pl.BlockSpec((_D,), full1),
        pl.BlockSpec((2 * _D, _D), full2),
        pl.BlockSpec((_D,), full1),
        pl.BlockSpec((_D,), full1),
        pl.BlockSpec((_D,), full1),
    ]
    return pl.pallas_call(
        _dense_body,
        grid=grid,
        in_specs=in_specs,
        out_specs=pl.BlockSpec((_BT, _D), row),
        out_shape=jax.ShapeDtypeStruct((_N, _D), jnp.float32),
        compiler_params=pltpu.CompilerParams(
            dimension_semantics=("parallel",)),
    )(acts2, ress2, num_arr, act_emb, res_emb,
      num_ln_g, num_ln_b, W1, b1, mlp_ln_g, mlp_ln_b,
      Wg, bg, Wb, bb, Wp, bp, proj_ln_g, proj_ln_b)


def kernel(activities, resources, num_arr, act_table, res_table,
           num_ln_g, num_ln_b, W1, b1, mlp_ln_g, mlp_ln_b,
           Wg, bg, Wb, bb, Wp, bp, proj_ln_g, proj_ln_b):
    acts = activities.astype(jnp.int32)
    ress = resources.astype(jnp.int32)
    act_emb, res_emb = _sc_gather(act_table, res_table, acts, ress)
    return _tc_dense(acts.reshape(_N, 1), ress.reshape(_N, 1), num_arr,
                     act_emb, res_emb,
                     num_ln_g, num_ln_b, W1, b1, mlp_ln_g, mlp_ln_b,
                     Wg, bg, Wb, bb, Wp, bp, proj_ln_g, proj_ln_b)
